# Initial kernel scaffold; baseline (speedup 1.0000x reference)
import jax
import jax.numpy as jnp
from jax.experimental import pallas as pl


def kernel(events, W, b):
    raise NotImplementedError("write your pallas kernel here")


# speedup vs baseline: 6.6688x; 6.6688x over previous
import functools

import jax
import jax.numpy as jnp
from jax.experimental import pallas as pl
from jax.experimental.pallas import tpu as pltpu

_INF = 3.0e38
_BIGI = 2**30


def _bf16_rne(x):
    b = jax.lax.bitcast_convert_type(x, jnp.int32)
    r = b + jnp.int32(0x7FFF) + ((b >> 16) & 1)
    r = r & jnp.int32(-65536)
    return jax.lax.bitcast_convert_type(r, jnp.float32)


def _edgeconv_body(N, F, K, R, ev_ref, cols_ref, sqr_ref, feats_ref, Wd_ref,
                   W2_ref, b_ref, out_ref, d2_s, msum_s):
    ev = ev_ref[0]
    feats_r = ev[:, :F]
    xr = _bf16_rne(ev[:, 0:1])
    yr = _bf16_rne(ev[:, 1:2])
    sqr = sqr_ref[0]
    xc = _bf16_rne(cols_ref[0, 0:1, :])
    yc = _bf16_rne(cols_ref[0, 1:2, :])
    sqc = cols_ref[0, 2:3, :]
    dot = xr * xc + yr * yc
    d2 = (sqr + sqc) - 2.0 * dot
    col = jax.lax.broadcasted_iota(jnp.int32, (R, N), 1)
    d2_s[...] = d2
    msum_s[...] = jnp.zeros((R, N), jnp.float32)

    def step(t, carry):
        d2v = d2_s[...]
        m = jnp.min(d2v, axis=1, keepdims=True)
        eq = d2v == m
        am = jnp.min(jnp.where(eq, col, _BIGI), axis=1, keepdims=True)
        sel = col == am
        keep = jnp.where(t > 0, 1.0, 0.0)
        msum_s[...] = msum_s[...] + sel.astype(jnp.float32) * keep
        d2_s[...] = jnp.where(sel, _INF, d2v)
        return carry

    jax.lax.fori_loop(0, K + 1, step, 0)

    mean = jax.lax.dot_general(
        msum_s[...], feats_ref[0], (((1,), (0,)), ((), ())),
        precision=jax.lax.Precision.HIGHEST,
        preferred_element_type=jnp.float32) * (1.0 / K)
    avg = (jax.lax.dot_general(feats_r, Wd_ref[...], (((1,), (0,)), ((), ())),
                               precision=jax.lax.Precision.HIGHEST,
                               preferred_element_type=jnp.float32)
           + jax.lax.dot_general(mean, W2_ref[...], (((1,), (0,)), ((), ())),
                                 precision=jax.lax.Precision.HIGHEST,
                                 preferred_element_type=jnp.float32)
           + b_ref[...])
    val = jnp.concatenate([avg, feats_r], axis=1)
    val = jnp.where(val >= 0.0, val, 0.2 * val)
    ones = jnp.ones((R, 1), jnp.float32)
    out_ref[0] = jnp.concatenate([val, ones], axis=1)


def _make_kernel(N, F, K, R, out_w, interpret=False):

    def run(events, W, b):
        B, n, fp1 = events.shape
        assert n == N and fp1 == F + 1
        x = events[:, :, 0]
        y = events[:, :, 1]
        sq = x * x + y * y
        cols = jnp.stack([x, y, sq], axis=1)
        sqr = sq[:, :, None]
        feats = events[:, :, :F]
        Wd = W[:F] - W[F:]
        W2 = W[F:]
        bb = b.reshape(1, -1)
        grid = (B, N // R)
        body = functools.partial(_edgeconv_body, N, F, K, R)
        return pl.pallas_call(
            body,
            grid=grid,
            in_specs=[
                pl.BlockSpec((1, R, F + 1), lambda bi, ri: (bi, ri, 0)),
                pl.BlockSpec((1, 3, N), lambda bi, ri: (bi, 0, 0)),
                pl.BlockSpec((1, R, 1), lambda bi, ri: (bi, ri, 0)),
                pl.BlockSpec((1, N, F), lambda bi, ri: (bi, 0, 0)),
                pl.BlockSpec((F, W.shape[1]), lambda bi, ri: (0, 0)),
                pl.BlockSpec((F, W.shape[1]), lambda bi, ri: (0, 0)),
                pl.BlockSpec((1, W.shape[1]), lambda bi, ri: (0, 0)),
            ],
            out_specs=pl.BlockSpec((1, R, out_w), lambda bi, ri: (bi, ri, 0)),
            out_shape=jax.ShapeDtypeStruct((B, N, out_w), jnp.float32),
            scratch_shapes=[
                pltpu.VMEM((R, N), jnp.float32),
                pltpu.VMEM((R, N), jnp.float32),
            ],
            interpret=interpret,
        )(events, cols, sqr, feats, Wd, W2, bb)

    return run


def kernel(events, W, b):
    F = events.shape[-1] - 1
    out_w = W.shape[1] + F + 1
    return _make_kernel(events.shape[1], F, 16, 256, out_w)(events, W, b)


# speedup vs baseline: 13.2620x; 1.9887x over previous
import functools

import jax
import jax.numpy as jnp
from jax.experimental import pallas as pl
from jax.experimental.pallas import tpu as pltpu

_INF = 3.0e38
_BIGI = 2**30


def _bf16_rne(x):
    b = jax.lax.bitcast_convert_type(x, jnp.int32)
    r = b + jnp.int32(0x7FFF) + ((b >> 16) & 1)
    r = r & jnp.int32(-65536)
    return jax.lax.bitcast_convert_type(r, jnp.float32)


def _edgeconv_body(N, F, K, R, ev_ref, cols_ref, sqr_ref, feats_ref, Wd_ref,
                   W2_ref, b_ref, out_ref, d2_s, msum_s):
    ev = ev_ref[0]
    feats_r = ev[:, :F]
    xr = _bf16_rne(ev[:, 0:1])
    yr = _bf16_rne(ev[:, 1:2])
    sqr = sqr_ref[0]
    xc = _bf16_rne(cols_ref[0, 0:1, :])
    yc = _bf16_rne(cols_ref[0, 1:2, :])
    sqc = cols_ref[0, 2:3, :]
    dot = xr * xc + yr * yc
    d2 = (sqr + sqc) - 2.0 * dot
    col = jax.lax.broadcasted_iota(jnp.int32, (R, N), 1)
    bits = jax.lax.bitcast_convert_type(d2, jnp.int32)
    skey = jnp.where(bits < 0, bits ^ jnp.int32(0x7FFFFFFF), bits)
    pk = (skey & jnp.int32(-2048)) | col
    d2_s[...] = pk
    msum_s[...] = pk

    m_first = None
    m_last = None
    for t in range(K + 1):
        pkv = d2_s[...]
        m = jnp.min(pkv, axis=1, keepdims=True)
        if t == 0:
            m_first = m
        m_last = m
        if t < K:
            d2_s[...] = jnp.where(pkv == m, jnp.int32(0x7FFFFFFF), pkv)

    pk0 = msum_s[...]
    sel = jnp.logical_and(pk0 <= m_last, pk0 != m_first)
    mask = sel.astype(jnp.float32)

    mean = jax.lax.dot_general(
        mask, feats_ref[0], (((1,), (0,)), ((), ())),
        precision=jax.lax.Precision.HIGHEST,
        preferred_element_type=jnp.float32) * (1.0 / K)
    avg = (jax.lax.dot_general(feats_r, Wd_ref[...], (((1,), (0,)), ((), ())),
                               precision=jax.lax.Precision.HIGHEST,
                               preferred_element_type=jnp.float32)
           + jax.lax.dot_general(mean, W2_ref[...], (((1,), (0,)), ((), ())),
                                 precision=jax.lax.Precision.HIGHEST,
                                 preferred_element_type=jnp.float32)
           + b_ref[...])
    val = jnp.concatenate([avg, feats_r], axis=1)
    val = jnp.where(val >= 0.0, val, 0.2 * val)
    ones = jnp.ones((R, 1), jnp.float32)
    out_ref[0] = jnp.concatenate([val, ones], axis=1)


def _make_kernel(N, F, K, R, out_w, interpret=False):

    def run(events, W, b):
        B, n, fp1 = events.shape
        assert n == N and fp1 == F + 1
        x = events[:, :, 0]
        y = events[:, :, 1]
        sq = x * x + y * y
        cols = jnp.stack([x, y, sq], axis=1)
        sqr = sq[:, :, None]
        feats = events[:, :, :F]
        Wd = W[:F] - W[F:]
        W2 = W[F:]
        bb = b.reshape(1, -1)
        grid = (B, N // R)
        body = functools.partial(_edgeconv_body, N, F, K, R)
        return pl.pallas_call(
            body,
            grid=grid,
            in_specs=[
                pl.BlockSpec((1, R, F + 1), lambda bi, ri: (bi, ri, 0)),
                pl.BlockSpec((1, 3, N), lambda bi, ri: (bi, 0, 0)),
                pl.BlockSpec((1, R, 1), lambda bi, ri: (bi, ri, 0)),
                pl.BlockSpec((1, N, F), lambda bi, ri: (bi, 0, 0)),
                pl.BlockSpec((F, W.shape[1]), lambda bi, ri: (0, 0)),
                pl.BlockSpec((F, W.shape[1]), lambda bi, ri: (0, 0)),
                pl.BlockSpec((1, W.shape[1]), lambda bi, ri: (0, 0)),
            ],
            out_specs=pl.BlockSpec((1, R, out_w), lambda bi, ri: (bi, ri, 0)),
            out_shape=jax.ShapeDtypeStruct((B, N, out_w), jnp.float32),
            scratch_shapes=[
                pltpu.VMEM((R, N), jnp.int32),
                pltpu.VMEM((R, N), jnp.int32),
            ],
            interpret=interpret,
        )(events, cols, sqr, feats, Wd, W2, bb)

    return run


def kernel(events, W, b):
    F = events.shape[-1] - 1
    out_w = W.shape[1] + F + 1
    return _make_kernel(events.shape[1], F, 16, 256, out_w)(events, W, b)


# speedup vs baseline: 14.1760x; 1.0689x over previous
import functools

import jax
import jax.numpy as jnp
from jax import lax
from jax.experimental import pallas as pl
from jax.experimental.pallas import tpu as pltpu
from jax.experimental.pallas import tpu_sc as plsc

_INF = 3.0e38
_BIGI = 2**30


def _bf16_rne(x):
    b = jax.lax.bitcast_convert_type(x, jnp.int32)
    r = b + jnp.int32(0x7FFF) + ((b >> 16) & 1)
    r = r & jnp.int32(-65536)
    return jax.lax.bitcast_convert_type(r, jnp.float32)


def _edgeconv_body(N, F, K, R, ev_ref, cols_ref, sqr_ref, feats_ref, Wd_ref,
                   W2_ref, b_ref, out_ref, d2_s, msum_s):
    ev = ev_ref[0]
    feats_r = ev[:, :F]
    xr = _bf16_rne(ev[:, 0:1])
    yr = _bf16_rne(ev[:, 1:2])
    sqr = sqr_ref[0]
    xc = _bf16_rne(cols_ref[0, 0:1, :])
    yc = _bf16_rne(cols_ref[0, 1:2, :])
    sqc = cols_ref[0, 2:3, :]
    dot = xr * xc + yr * yc
    d2 = (sqr + sqc) - 2.0 * dot
    col = jax.lax.broadcasted_iota(jnp.int32, (R, N), 1)
    bits = jax.lax.bitcast_convert_type(d2, jnp.int32)
    skey = jnp.where(bits < 0, bits ^ jnp.int32(0x7FFFFFFF), bits)
    pk = (skey & jnp.int32(-2048)) | col
    d2_s[...] = pk
    msum_s[...] = pk

    m_first = None
    m_last = None
    for t in range(K + 1):
        pkv = d2_s[...]
        m = jnp.min(pkv, axis=1, keepdims=True)
        if t == 0:
            m_first = m
        m_last = m
        if t < K:
            d2_s[...] = jnp.where(pkv == m, jnp.int32(0x7FFFFFFF), pkv)

    pk0 = msum_s[...]
    sel = jnp.logical_and(pk0 <= m_last, pk0 != m_first)
    mask = sel.astype(jnp.float32)

    mean = jax.lax.dot_general(
        mask, feats_ref[0], (((1,), (0,)), ((), ())),
        precision=jax.lax.Precision.HIGHEST,
        preferred_element_type=jnp.float32) * (1.0 / K)
    avg = (jax.lax.dot_general(feats_r, Wd_ref[...], (((1,), (0,)), ((), ())),
                               precision=jax.lax.Precision.HIGHEST,
                               preferred_element_type=jnp.float32)
           + jax.lax.dot_general(mean, W2_ref[...], (((1,), (0,)), ((), ())),
                                 precision=jax.lax.Precision.HIGHEST,
                                 preferred_element_type=jnp.float32)
           + b_ref[...])
    val = jnp.concatenate([avg, feats_r], axis=1)
    val = jnp.where(val >= 0.0, val, 0.2 * val)
    ones = jnp.ones((R, 1), jnp.float32)
    out_ref[0] = jnp.concatenate([val, ones], axis=1)


def _make_kernel(N, F, K, R, out_w, interpret=False):

    def run(events, W, b):
        B, n, fp1 = events.shape
        assert n == N and fp1 == F + 1
        x = events[:, :, 0]
        y = events[:, :, 1]
        sq = x * x + y * y
        cols = jnp.stack([x, y, sq], axis=1)
        sqr = sq[:, :, None]
        feats = events[:, :, :F]
        Wd = W[:F] - W[F:]
        W2 = W[F:]
        bb = b.reshape(1, -1)
        grid = (B, N // R)
        body = functools.partial(_edgeconv_body, N, F, K, R)
        return pl.pallas_call(
            body,
            grid=grid,
            in_specs=[
                pl.BlockSpec((1, R, F + 1), lambda bi, ri: (bi, ri, 0)),
                pl.BlockSpec((1, 3, N), lambda bi, ri: (bi, 0, 0)),
                pl.BlockSpec((1, R, 1), lambda bi, ri: (bi, ri, 0)),
                pl.BlockSpec((1, N, F), lambda bi, ri: (bi, 0, 0)),
                pl.BlockSpec((F, W.shape[1]), lambda bi, ri: (0, 0)),
                pl.BlockSpec((F, W.shape[1]), lambda bi, ri: (0, 0)),
                pl.BlockSpec((1, W.shape[1]), lambda bi, ri: (0, 0)),
            ],
            out_specs=pl.BlockSpec((1, R, out_w), lambda bi, ri: (bi, ri, 0)),
            out_shape=jax.ShapeDtypeStruct((B, N, out_w), jnp.float32),
            scratch_shapes=[
                pltpu.VMEM((R, N), jnp.int32),
                pltpu.VMEM((R, N), jnp.int32),
            ],
            interpret=interpret,
        )(events, cols, sqr, feats, Wd, W2, bb)

    return run


_IMAX = 0x7FFFFFFF


def _make_sc_knn_mean(E, N, F, K):
    NW = 32
    L = 16
    CH = N // L
    RPT = E * N // NW
    TPE = N // RPT
    mesh = plsc.VectorSubcoreMesh(core_axis_name="c", subcore_axis_name="s")

    @functools.partial(
        pl.kernel,
        out_type=jax.ShapeDtypeStruct((E, N * F), jnp.float32),
        mesh=mesh,
        compiler_params=pltpu.CompilerParams(needs_layout_passes=False),
        scratch_types=[
            pltpu.VMEM((N,), jnp.float32),
            pltpu.VMEM((N,), jnp.float32),
            pltpu.VMEM((N,), jnp.float32),
            pltpu.VMEM((N * F,), jnp.float32),
            pltpu.VMEM((RPT * F,), jnp.float32),
        ],
    )
    def knn(xb_h, yb_h, sq_h, feats_h, mean_h, xb_v, yb_v, sq_v, feats_v,
            out_v):
        wid = lax.axis_index("s") * 2 + lax.axis_index("c")
        e = wid // TPE
        r0 = (wid % TPE) * RPT
        pltpu.sync_copy(xb_h.at[e], xb_v)
        pltpu.sync_copy(yb_h.at[e], yb_v)
        pltpu.sync_copy(sq_h.at[e], sq_v)
        pltpu.sync_copy(feats_h.at[e], feats_v)

        lane = lax.iota(jnp.int32, L)
        imax_v = jnp.full((L,), _IMAX, jnp.int32)

        def group_body(q, carry):
            rbase = r0 + q * L
            rx = xb_v[pl.ds(rbase, L)]
            ry = yb_v[pl.ds(rbase, L)]
            rsq = sq_v[pl.ds(rbase, L)]
            for i in range(L):
                xi = rx[i]
                yi = ry[i]
                sqi = rsq[i]

                def chunk_body(c, st):
                    best, g1, thr = st
                    base = c * L
                    kx = xb_v[pl.ds(base, L)]
                    ky = yb_v[pl.ds(base, L)]
                    ksq = sq_v[pl.ds(base, L)]
                    dot = xi * kx + yi * ky
                    d2 = (sqi + ksq) - 2.0 * dot
                    bits = lax.bitcast_convert_type(d2, jnp.int32)
                    skey = jnp.where(bits < 0, bits ^ jnp.int32(_IMAX), bits)
                    pk = (skey & jnp.int32(-2048)) | (base + lane)
                    psort = plsc.sort_key_val(pk, pk)[0]

                    def merge(_):
                        cdesc = lax.rev(psort, (0,))
                        ls = plsc.sort_key_val(
                            jnp.minimum(best, cdesc), lane)[0]
                        l0 = ls[0]
                        new_g = jnp.minimum(g1, l0)
                        repl = jnp.maximum(g1, l0)
                        best2 = plsc.sort_key_val(
                            jnp.where(lane == 0, repl, ls), lane)[0]
                        return best2, new_g, best2[L - 1]

                    return lax.cond(psort[0] < thr, merge, lambda _: st, 0)

                init = (imax_v, jnp.int32(_IMAX), jnp.int32(_IMAX))
                best, _, _ = lax.fori_loop(0, CH, chunk_body, init)
                bidx = best & jnp.int32(2047)

                acc = jnp.zeros((L,), jnp.float32)
                for t in range(K):
                    acc = acc + feats_v[pl.ds(bidx[t] * F, F)]
                out_v[pl.ds((q * L + i) * F, F)] = acc * (1.0 / K)
            return carry

        lax.fori_loop(0, RPT // L, group_body, 0)
        pltpu.sync_copy(out_v, mean_h.at[e, pl.ds(r0 * F, RPT * F)])

    return knn


def _post_body(F, ev_ref, mean_ref, Wd_ref, W2_ref, b_ref, out_ref):
    ev = ev_ref[0]
    feats_r = ev[:, :F]
    avg = (jax.lax.dot_general(feats_r, Wd_ref[...], (((1,), (0,)), ((), ())),
                               precision=jax.lax.Precision.HIGHEST,
                               preferred_element_type=jnp.float32)
           + jax.lax.dot_general(mean_ref[0], W2_ref[...],
                                 (((1,), (0,)), ((), ())),
                                 precision=jax.lax.Precision.HIGHEST,
                                 preferred_element_type=jnp.float32)
           + b_ref[...])
    val = jnp.concatenate([avg, feats_r], axis=1)
    val = jnp.where(val >= 0.0, val, 0.2 * val)
    R = val.shape[0]
    out_ref[0] = jnp.concatenate([val, jnp.ones((R, 1), jnp.float32)], axis=1)


def _make_post(N, F, R, out_w):
    def run(events_sc, means, W, b):
        E = events_sc.shape[0]
        Wd = W[:F] - W[F:]
        W2 = W[F:]
        bb = b.reshape(1, -1)
        body = functools.partial(_post_body, F)
        return pl.pallas_call(
            body,
            grid=(E, N // R),
            in_specs=[
                pl.BlockSpec((1, R, F + 1), lambda bi, ri: (bi, ri, 0)),
                pl.BlockSpec((1, R, F), lambda bi, ri: (bi, ri, 0)),
                pl.BlockSpec((F, W.shape[1]), lambda bi, ri: (0, 0)),
                pl.BlockSpec((F, W.shape[1]), lambda bi, ri: (0, 0)),
                pl.BlockSpec((1, W.shape[1]), lambda bi, ri: (0, 0)),
            ],
            out_specs=pl.BlockSpec((1, R, out_w), lambda bi, ri: (bi, ri, 0)),
            out_shape=jax.ShapeDtypeStruct((E, N, out_w), jnp.float32),
        )(events_sc, means, Wd, W2, bb)

    return run


_E_SC = 2


def kernel(events, W, b):
    B, N, fp1 = events.shape
    F = fp1 - 1
    K = 16
    out_w = W.shape[1] + F + 1
    ev_sc = events[:_E_SC]
    x = ev_sc[:, :, 0]
    y = ev_sc[:, :, 1]
    sq = x * x + y * y
    xb = _bf16_rne(x)
    yb = _bf16_rne(y)
    feats_sc = ev_sc[:, :, :F]
    means = _make_sc_knn_mean(_E_SC, N, F, K)(
        xb, yb, sq, feats_sc.reshape(_E_SC, N * F))
    out_sc = _make_post(N, F, 256, out_w)(ev_sc, means.reshape(_E_SC, N, F),
                                          W, b)
    out_tc = _make_kernel(N, F, K, 256, out_w)(events[_E_SC:], W, b)
    return jnp.concatenate([out_sc, out_tc], axis=0)
